# baseline (speedup 1.0000x reference)
import functools

import jax
import jax.numpy as jnp
from jax import lax
from jax.experimental import pallas as pl
from jax.experimental.pallas import tpu as pltpu
from jax.experimental.pallas import tpu_sc as plsc

N_NODES = 50000
MAX_CONN = 10
M = 128
NWORD = 4
T = 32
NP = 51200
NC = NP // 8
ROWS4 = NP * MAX_CONN * NWORD
NW = 32
RPW = ROWS4 // NW
GPW = RPW // 128
NBUF = 5
NBC = 128


def _sc_gather(state_flat, idx3):
    mesh = plsc.VectorSubcoreMesh(core_axis_name="c", subcore_axis_name="s")

    @functools.partial(
        pl.kernel,
        mesh=mesh,
        out_type=jax.ShapeDtypeStruct((ROWS4,), jnp.int32),
        scratch_types=[
            pltpu.VMEM((GPW, 128), jnp.int32),
            pltpu.VMEM_SHARED((NWORD * NP,), jnp.int32),
        ]
        + [pltpu.VMEM((128,), jnp.int32) for _ in range(NBUF)]
        + [pltpu.SemaphoreType.DMA for _ in range(2 * NBUF)],
    )
    def k(state_hbm, idx_hbm, out_hbm, idx_v, shared, *bufs_sems):
        bufs = bufs_sems[:NBUF]
        gsem = bufs_sems[NBUF : 2 * NBUF]
        wsem = bufs_sems[2 * NBUF : 3 * NBUF]
        wid = lax.axis_index("s") * 2 + lax.axis_index("c")
        sid = lax.axis_index("s")
        seg = NWORD * NP // 16
        pltpu.sync_copy(state_hbm.at[pl.ds(sid * seg, seg)],
                        shared.at[pl.ds(sid * seg, seg)])
        pltpu.sync_copy(idx_hbm.at[wid], idx_v)
        plsc.subcore_barrier()
        state_src = shared
        base = wid * RPW

        def out_at(g):
            return out_hbm.at[pl.ds(base + g * 128, 128)]

        for b in range(NBUF):
            pltpu.async_copy(state_src.at[idx_v.at[b]], bufs[b], gsem[b])

        @pl.loop(1, GPW // NBUF)
        def _(o):
            for b in range(NBUF):
                g = o * NBUF + b
                pltpu.make_async_copy(state_src.at[idx_v.at[g - NBUF]],
                                      bufs[b], gsem[b]).wait()
                pltpu.async_copy(bufs[b], out_at(g - NBUF), wsem[b])
            for b in range(NBUF):
                g = o * NBUF + b
                pltpu.make_async_copy(bufs[b], out_at(g - NBUF), wsem[b]).wait()
                pltpu.async_copy(state_src.at[idx_v.at[g]], bufs[b], gsem[b])

        for b in range(NBUF):
            g = GPW - NBUF + b
            pltpu.make_async_copy(state_src.at[idx_v.at[g]],
                                  bufs[b], gsem[b]).wait()
            pltpu.sync_copy(bufs[b], out_at(g))

    return k(state_flat, idx3)


def _mux(s, a, b):
    return b ^ (s & (a ^ b))


def _tc_step_body(g_ref, lut_ref, x_ref, out_ref):
    B = [[g_ref[k, w] for w in range(NWORD)] for k in range(MAX_CONN)]
    stack = []
    for h in range(32):
        lh = lut_ref[h]
        leaves = [(lh << (31 - jj)) >> 31 for jj in range(32)]
        roots = []
        for w in range(NWORD):
            cur = leaves
            for lev in range(5):
                s = B[lev][w]
                cur = [_mux(s, cur[2 * i + 1], cur[2 * i])
                       for i in range(len(cur) // 2)]
            roots.append(cur[0])
        node = (5, roots)
        while stack and stack[-1][0] == node[0]:
            ph, proots = stack.pop()
            node = (ph + 1,
                    [_mux(B[ph][w], node[1][w], proots[w])
                     for w in range(NWORD)])
        stack.append(node)
    (_, final), = stack
    for w in range(NWORD):
        out_ref[w] = final[w] ^ x_ref[w]


def _tc_step(g4, lutp, xnext):
    return pl.pallas_call(
        _tc_step_body,
        grid=(NC // NBC,),
        in_specs=[
            pl.BlockSpec((MAX_CONN, NWORD, 8, NBC), lambda i: (0, 0, 0, i)),
            pl.BlockSpec((32, 8, NBC), lambda i: (0, 0, i)),
            pl.BlockSpec((NWORD, 8, NBC), lambda i: (0, 0, i)),
        ],
        out_specs=pl.BlockSpec((NWORD, 8, NBC), lambda i: (0, 0, i)),
        out_shape=jax.ShapeDtypeStruct((NWORD, 8, NC), jnp.int32),
    )(g4, lutp, xnext)


def _tc_xor_body(a_ref, b_ref, o_ref):
    o_ref[...] = a_ref[...] ^ b_ref[...]


def _tc_xor(a, b):
    return pl.pallas_call(
        _tc_xor_body,
        grid=(NC // 1280,),
        in_specs=[
            pl.BlockSpec((NWORD, 8, 1280), lambda i: (0, 0, i)),
            pl.BlockSpec((NWORD, 8, 1280), lambda i: (0, 0, i)),
        ],
        out_specs=pl.BlockSpec((NWORD, 8, 1280), lambda i: (0, 0, i)),
        out_shape=jax.ShapeDtypeStruct((NWORD, 8, NC), jnp.int32),
    )(a, b)


def _tc_readout_body(s_ref, w_ref, o_ref):
    for i in range(2):
        for w in range(NWORD):
            plane = s_ref[w]
            for bit in range(32):
                v = ((plane >> bit) & 1).astype(jnp.float32) * w_ref[i]
                mm = 32 * w + bit
                o_ref[i : i + 1, mm : mm + 1] = jnp.sum(v).reshape(1, 1)


def _tc_readout(sp, wp):
    return pl.pallas_call(
        _tc_readout_body,
        in_specs=[
            pl.BlockSpec((NWORD, 8, NC), lambda: (0, 0, 0)),
            pl.BlockSpec((2, 8, NC), lambda: (0, 0, 0)),
        ],
        out_specs=pl.BlockSpec((8, M), lambda: (0, 0)),
        out_shape=jax.ShapeDtypeStruct((8, M), jnp.float32),
    )(sp, wp)


def kernel(x, lut, adj_list, adj_mask, input_nodes, init_states, W, b):
    m = x.shape[0]
    adj = adj_list.astype(jnp.int32)
    pad_ids = (jnp.arange(NP * MAX_CONN, dtype=jnp.int32) % (NP - N_NODES)
               + N_NODES).reshape(NP, MAX_CONN)
    a2 = jnp.where(adj_mask, adj, pad_ids[:N_NODES])
    a2 = jnp.concatenate([a2, pad_ids[N_NODES:]], axis=0)
    a2k = a2.T.reshape(MAX_CONN, 1, NP)
    woff = (jnp.arange(NWORD, dtype=jnp.int32) * NP).reshape(1, NWORD, 1)
    idx3 = (a2k + woff).reshape(NW, GPW, 128)

    powers = jnp.uint32(1) << jnp.arange(32, dtype=jnp.uint32)
    lutw = (lut.astype(jnp.uint32).reshape(N_NODES, 32, 32) * powers).sum(
        axis=-1, dtype=jnp.uint32)
    lutw = lax.bitcast_convert_type(lutw, jnp.int32)
    lutw = jnp.pad(lutw, ((0, NP - N_NODES), (0, 0)))
    lutp = lutw.T.reshape(32, 8, NC)

    xb = jnp.transpose(x.reshape(m, T, 32).astype(jnp.uint32), (1, 2, 0))
    xw = (xb.reshape(T, 32, NWORD, 32) * powers).sum(axis=-1,
                                                     dtype=jnp.uint32)
    xw = lax.bitcast_convert_type(xw, jnp.int32)
    xp = jnp.zeros((T + 1, NWORD, NP), jnp.int32)
    xp = xp.at[:T, :, input_nodes].set(jnp.transpose(xw, (0, 2, 1)))
    xp = xp.reshape(T + 1, NWORD, 8, NC)

    ini = (init_states.T.astype(jnp.uint32).reshape(N_NODES, NWORD, 32)
           * powers).sum(axis=-1, dtype=jnp.uint32)
    ini = lax.bitcast_convert_type(ini, jnp.int32)
    ini = jnp.pad(ini, ((0, NP - N_NODES), (0, 0)))
    ini = ini.T.reshape(NWORD, 8, NC)

    wp = jnp.pad(W, ((0, 0), (0, NP - N_NODES))).reshape(2, 8, NC)

    s0 = _tc_xor(ini, xp[0])

    def body(s, xnext):
        g = _sc_gather(s.reshape(NWORD * NP), idx3)
        g4 = g.reshape(MAX_CONN, NWORD, 8, NC)
        return _tc_step(g4, lutp, xnext), None

    s_final, _ = lax.scan(body, s0, xp[1:], unroll=True)

    acc = _tc_readout(s_final, wp)
    out = acc[: W.shape[0], :m].T + b[None, :]
    return out


# speedup vs baseline: 1.5882x; 1.5882x over previous
import functools

import jax
import jax.numpy as jnp
from jax import lax
from jax.experimental import pallas as pl
from jax.experimental.pallas import tpu as pltpu
from jax.experimental.pallas import tpu_sc as plsc

N_NODES = 50000
MAX_CONN = 10
M = 128
NWORD = 4
T = 32
NP = 51200
NC = NP // 8
ROWS4 = NP * MAX_CONN * NWORD
NW = 32
RPW = ROWS4 // NW
GPW = RPW // 128
NBUF = 5
NBC = 128


def _sc_gather(state_flat, idx3):
    mesh = plsc.VectorSubcoreMesh(core_axis_name="c", subcore_axis_name="s")

    @functools.partial(
        pl.kernel,
        mesh=mesh,
        out_type=jax.ShapeDtypeStruct((ROWS4,), jnp.int32),
        scratch_types=[
            pltpu.VMEM((GPW, 128), jnp.int32),
            pltpu.VMEM_SHARED((NWORD * NP,), jnp.int32),
        ]
        + [pltpu.VMEM((128,), jnp.int32) for _ in range(NBUF)]
        + [pltpu.SemaphoreType.DMA for _ in range(2 * NBUF)],
    )
    def k(state_hbm, idx_hbm, out_hbm, idx_v, shared, *bufs_sems):
        bufs = bufs_sems[:NBUF]
        gsem = bufs_sems[NBUF : 2 * NBUF]
        wsem = bufs_sems[2 * NBUF : 3 * NBUF]
        wid = lax.axis_index("s") * 2 + lax.axis_index("c")
        sid = lax.axis_index("s")
        seg = NWORD * NP // 16
        pltpu.sync_copy(state_hbm.at[pl.ds(sid * seg, seg)],
                        shared.at[pl.ds(sid * seg, seg)])
        pltpu.sync_copy(idx_hbm.at[wid], idx_v)
        plsc.subcore_barrier()
        state_src = shared
        base = wid * RPW

        def out_at(g):
            return out_hbm.at[pl.ds(base + g * 128, 128)]

        for b in range(NBUF):
            pltpu.async_copy(state_src.at[idx_v.at[b]], bufs[b], gsem[b])

        @pl.loop(1, GPW // NBUF)
        def _(o):
            for b in range(NBUF):
                g = o * NBUF + b
                pltpu.make_async_copy(state_src.at[idx_v.at[g - NBUF]],
                                      bufs[b], gsem[b]).wait()
                pltpu.async_copy(bufs[b], out_at(g - NBUF), wsem[b])
            for b in range(NBUF):
                g = o * NBUF + b
                pltpu.make_async_copy(bufs[b], out_at(g - NBUF), wsem[b]).wait()
                pltpu.async_copy(state_src.at[idx_v.at[g]], bufs[b], gsem[b])

        for b in range(NBUF):
            g = GPW - NBUF + b
            pltpu.make_async_copy(state_src.at[idx_v.at[g]],
                                  bufs[b], gsem[b]).wait()
            pltpu.sync_copy(bufs[b], out_at(g))

    return k(state_flat, idx3)


def _mux(s, a, b):
    return b ^ (s & (a ^ b))


def _tc_step_body(g_ref, lut_ref, x_ref, out_ref):
    B = [[g_ref[k, w] for w in range(NWORD)] for k in range(MAX_CONN)]
    stack = []
    for h in range(32):
        lh = lut_ref[h]
        leaves = [(lh << (31 - jj)) >> 31 for jj in range(32)]
        roots = []
        for w in range(NWORD):
            cur = leaves
            for lev in range(5):
                s = B[lev][w]
                cur = [_mux(s, cur[2 * i + 1], cur[2 * i])
                       for i in range(len(cur) // 2)]
            roots.append(cur[0])
        node = (5, roots)
        while stack and stack[-1][0] == node[0]:
            ph, proots = stack.pop()
            node = (ph + 1,
                    [_mux(B[ph][w], node[1][w], proots[w])
                     for w in range(NWORD)])
        stack.append(node)
    (_, final), = stack
    for w in range(NWORD):
        out_ref[w] = final[w] ^ x_ref[w]


def _tc_step(g4, lutp, xnext):
    return pl.pallas_call(
        _tc_step_body,
        grid=(NC // NBC,),
        in_specs=[
            pl.BlockSpec((MAX_CONN, NWORD, 8, NBC), lambda i: (0, 0, 0, i)),
            pl.BlockSpec((32, 8, NBC), lambda i: (0, 0, i)),
            pl.BlockSpec((NWORD, 8, NBC), lambda i: (0, 0, i)),
        ],
        out_specs=pl.BlockSpec((NWORD, 8, NBC), lambda i: (0, 0, i)),
        out_shape=jax.ShapeDtypeStruct((NWORD, 8, NC), jnp.int32),
    )(g4, lutp, xnext)


def _tc_xor_body(a_ref, b_ref, o_ref):
    o_ref[...] = a_ref[...] ^ b_ref[...]


def _tc_xor(a, b):
    return pl.pallas_call(
        _tc_xor_body,
        grid=(NC // 1280,),
        in_specs=[
            pl.BlockSpec((NWORD, 8, 1280), lambda i: (0, 0, i)),
            pl.BlockSpec((NWORD, 8, 1280), lambda i: (0, 0, i)),
        ],
        out_specs=pl.BlockSpec((NWORD, 8, 1280), lambda i: (0, 0, i)),
        out_shape=jax.ShapeDtypeStruct((NWORD, 8, NC), jnp.int32),
    )(a, b)


def _tc_readout_body(s_ref, w_ref, o_ref):
    for i in range(2):
        for w in range(NWORD):
            plane = s_ref[w]
            for bit in range(32):
                v = ((plane >> bit) & 1).astype(jnp.float32) * w_ref[i]
                mm = 32 * w + bit
                o_ref[i : i + 1, mm : mm + 1] = jnp.sum(v).reshape(1, 1)


def _tc_readout(sp, wp):
    return pl.pallas_call(
        _tc_readout_body,
        in_specs=[
            pl.BlockSpec((NWORD, 8, NC), lambda: (0, 0, 0)),
            pl.BlockSpec((2, 8, NC), lambda: (0, 0, 0)),
        ],
        out_specs=pl.BlockSpec((8, M), lambda: (0, 0)),
        out_shape=jax.ShapeDtypeStruct((8, M), jnp.float32),
    )(sp, wp)


def kernel(x, lut, adj_list, adj_mask, input_nodes, init_states, W, b):
    m = x.shape[0]
    adj = adj_list.astype(jnp.int32)
    pad_ids = (jnp.arange(NP * MAX_CONN, dtype=jnp.int32) % (NP - N_NODES)
               + N_NODES).reshape(NP, MAX_CONN)
    a2 = jnp.where(adj_mask, adj, pad_ids[:N_NODES])
    a2 = jnp.concatenate([a2, pad_ids[N_NODES:]], axis=0)
    a2k = a2.T.reshape(MAX_CONN, 1, NP)
    woff = (jnp.arange(NWORD, dtype=jnp.int32) * NP).reshape(1, NWORD, 1)
    idx3 = (a2k + woff).reshape(NW, GPW, 128)

    powers = jnp.uint32(1) << jnp.arange(32, dtype=jnp.uint32)
    lutw = (lut.astype(jnp.uint32).reshape(N_NODES, 32, 32) * powers).sum(
        axis=-1, dtype=jnp.uint32)
    lutw = lax.bitcast_convert_type(lutw, jnp.int32)
    lutw = jnp.pad(lutw, ((0, NP - N_NODES), (0, 0)))
    lutp = lutw.T.reshape(32, 8, NC)

    xb = jnp.transpose(x.reshape(m, T, 32).astype(jnp.uint32), (1, 2, 0))
    xw = (xb.reshape(T, 32, NWORD, 32) * powers).sum(axis=-1,
                                                     dtype=jnp.uint32)
    xw = lax.bitcast_convert_type(xw, jnp.int32)
    xp = jnp.zeros((T + 1, NWORD, NP), jnp.int32)
    xp = xp.at[:T, :, input_nodes].set(jnp.transpose(xw, (0, 2, 1)))
    xp = xp.reshape(T + 1, NWORD, 8, NC)

    ini = (init_states.T.astype(jnp.uint32).reshape(N_NODES, NWORD, 32)
           * powers).sum(axis=-1, dtype=jnp.uint32)
    ini = lax.bitcast_convert_type(ini, jnp.int32)
    ini = jnp.pad(ini, ((0, NP - N_NODES), (0, 0)))
    ini = ini.T.reshape(NWORD, 8, NC)

    wp = jnp.pad(W, ((0, 0), (0, NP - N_NODES))).reshape(2, 8, NC)

    s0 = _tc_xor(ini, xp[0])

    def body(s, xnext):
        g = _sc_gather(s.reshape(NWORD * NP), idx3)
        g4 = g.reshape(MAX_CONN, NWORD, 8, NC)
        return _tc_step(g4, lutp, xnext), None

    s_final, _ = lax.scan(body, s0, xp[1:])

    acc = _tc_readout(s_final, wp)
    out = acc[: W.shape[0], :m].T + b[None, :]
    return out
